# baseline (speedup 1.0000x reference)
import functools

import jax
import jax.numpy as jnp
from jax.experimental import pallas as pl
from jax.experimental.pallas import tpu as pltpu

_EPS = 1e-12


def _prep_body(fw_ref, w1_ref, mn_ref, vp_ref, *, c):
    fw = fw_ref[...]
    m = fw[:, :c]
    n1 = jnp.sqrt(jnp.sum(m * m, axis=1, keepdims=True))
    mn_ref[...] = (m / jnp.maximum(n1, _EPS)).astype(jnp.bfloat16)
    n2 = jnp.sqrt(jnp.sum(fw * fw, axis=1, keepdims=True))
    fwn = fw / jnp.maximum(n2, _EPS)
    vp_ref[...] = jnp.dot(
        fwn, w1_ref[...], preferred_element_type=jnp.float32
    ).astype(jnp.bfloat16)


def _main_body(xq_ref, mn_ref, vp_ref, b1_ref, w2_ref, b2_ref, out_ref,
               xn_ref, acc_ref, den_ref, *, nk):
    j = pl.program_id(1)

    @pl.when(j == 0)
    def _init():
        xq = xq_ref[...]
        nrm = jnp.sqrt(jnp.sum(xq * xq, axis=1, keepdims=True))
        xn_ref[...] = (xq / jnp.maximum(nrm, _EPS)).astype(jnp.bfloat16)
        acc_ref[...] = jnp.zeros_like(acc_ref)
        den_ref[...] = jnp.zeros_like(den_ref)

    xn = xn_ref[...]
    s = jax.lax.dot_general(
        xn, mn_ref[...], (((1,), (1,)), ((), ())),
        preferred_element_type=jnp.float32)
    e = jnp.exp(s)
    acc_ref[...] += jnp.dot(e.astype(jnp.bfloat16), vp_ref[...],
                            preferred_element_type=jnp.float32)
    den_ref[...] += jnp.broadcast_to(
        jnp.sum(e, axis=1, keepdims=True), den_ref.shape)

    @pl.when(j == nk - 1)
    def _epilogue():
        den = den_ref[...][:, :1]
        o = acc_ref[...] / den + b1_ref[...]
        h1 = 0.5 * o * (1.0 + jax.lax.erf(o * (2.0 ** -0.5)))
        out_ref[...] = (jnp.dot(h1, w2_ref[...], preferred_element_type=jnp.float32)
                        + b2_ref[...])


def kernel(x, feat_w, W1, b1, W2, b2):
    b, c, h, w = x.shape
    n = b * h * w
    k, c4 = feat_w.shape
    hdim = W2.shape[1]

    x_flat = jnp.transpose(x, (0, 2, 3, 1)).reshape(n, c)
    b1_2d = b1.reshape(1, hdim)
    b2_2d = b2.reshape(1, hdim)

    bkp = 1024
    nkp = k // bkp
    mn, vp = pl.pallas_call(
        functools.partial(_prep_body, c=c),
        grid=(nkp,),
        in_specs=[
            pl.BlockSpec((bkp, c4), lambda i: (i, 0)),
            pl.BlockSpec((c4, hdim), lambda i: (0, 0)),
        ],
        out_specs=[
            pl.BlockSpec((bkp, c), lambda i: (i, 0)),
            pl.BlockSpec((bkp, hdim), lambda i: (i, 0)),
        ],
        out_shape=[
            jax.ShapeDtypeStruct((k, c), jnp.bfloat16),
            jax.ShapeDtypeStruct((k, hdim), jnp.bfloat16),
        ],
    )(feat_w, W1)

    bq, bk = 256, 8192
    nq, nk = n // bq, k // bk
    out2d = pl.pallas_call(
        functools.partial(_main_body, nk=nk),
        grid=(nq, nk),
        in_specs=[
            pl.BlockSpec((bq, c), lambda i, j: (i, 0)),
            pl.BlockSpec((bk, c), lambda i, j: (j, 0)),
            pl.BlockSpec((bk, hdim), lambda i, j: (j, 0)),
            pl.BlockSpec((1, hdim), lambda i, j: (0, 0)),
            pl.BlockSpec((hdim, hdim), lambda i, j: (0, 0)),
            pl.BlockSpec((1, hdim), lambda i, j: (0, 0)),
        ],
        out_specs=pl.BlockSpec((bq, hdim), lambda i, j: (i, 0)),
        out_shape=jax.ShapeDtypeStruct((n, hdim), jnp.float32),
        scratch_shapes=[
            pltpu.VMEM((bq, c), jnp.bfloat16),
            pltpu.VMEM((bq, hdim), jnp.float32),
            pltpu.VMEM((bq, 128), jnp.float32),
        ],
        compiler_params=pltpu.CompilerParams(
            dimension_semantics=("parallel", "arbitrary")),
    )(x_flat, mn, vp, b1_2d, W2, b2_2d)

    return jnp.transpose(out2d.reshape(b, h, w, hdim), (0, 3, 1, 2))


# speedup vs baseline: 1.1391x; 1.1391x over previous
import functools

import jax
import jax.numpy as jnp
from jax.experimental import pallas as pl
from jax.experimental.pallas import tpu as pltpu

_EPS = 1e-12


def _prep_body(fw_ref, w1_ref, mn_ref, vp_ref, *, c):
    fw = fw_ref[...]
    m = fw[:, :c]
    n1 = jnp.sqrt(jnp.sum(m * m, axis=1, keepdims=True))
    mn_ref[...] = (m / jnp.maximum(n1, _EPS)).astype(jnp.bfloat16)
    n2 = jnp.sqrt(jnp.sum(fw * fw, axis=1, keepdims=True))
    fwn = fw / jnp.maximum(n2, _EPS)
    vp_ref[...] = jnp.dot(
        fwn, w1_ref[...], preferred_element_type=jnp.float32
    ).astype(jnp.bfloat16)


def _main_body(xq_ref, mn_ref, vp_ref, b1_ref, w2_ref, b2_ref, out_ref,
               xn_ref, acc_ref, den_ref, *, nk):
    j = pl.program_id(1)

    @pl.when(j == 0)
    def _init():
        xq = xq_ref[...]
        nrm = jnp.sqrt(jnp.sum(xq * xq, axis=1, keepdims=True))
        xn_ref[...] = (xq / jnp.maximum(nrm, _EPS)).astype(jnp.bfloat16)
        acc_ref[...] = jnp.zeros_like(acc_ref)
        den_ref[...] = jnp.zeros_like(den_ref)

    xn = xn_ref[...]
    s = jax.lax.dot_general(
        xn, mn_ref[...], (((1,), (1,)), ((), ())),
        preferred_element_type=jnp.float32)
    e = jnp.exp(s)
    acc_ref[...] += jnp.dot(e.astype(jnp.bfloat16), vp_ref[...],
                            preferred_element_type=jnp.float32)
    den_ref[...] += jnp.broadcast_to(
        jnp.sum(e, axis=1, keepdims=True), den_ref.shape)

    @pl.when(j == nk - 1)
    def _epilogue():
        den = den_ref[...][:, :1]
        o = acc_ref[...] / den + b1_ref[...]
        h1 = 0.5 * o * (1.0 + jax.lax.erf(o * (2.0 ** -0.5)))
        out_ref[...] = (jnp.dot(h1, w2_ref[...], preferred_element_type=jnp.float32)
                        + b2_ref[...])


def kernel(x, feat_w, W1, b1, W2, b2):
    b, c, h, w = x.shape
    n = b * h * w
    k, c4 = feat_w.shape
    hdim = W2.shape[1]

    x_flat = jnp.transpose(x, (0, 2, 3, 1)).reshape(n, c)
    b1_2d = b1.reshape(1, hdim)
    b2_2d = b2.reshape(1, hdim)

    bkp = 1024
    nkp = k // bkp
    mn, vp = pl.pallas_call(
        functools.partial(_prep_body, c=c),
        grid=(nkp,),
        in_specs=[
            pl.BlockSpec((bkp, c4), lambda i: (i, 0)),
            pl.BlockSpec((c4, hdim), lambda i: (0, 0)),
        ],
        out_specs=[
            pl.BlockSpec((bkp, c), lambda i: (i, 0)),
            pl.BlockSpec((bkp, hdim), lambda i: (i, 0)),
        ],
        out_shape=[
            jax.ShapeDtypeStruct((k, c), jnp.bfloat16),
            jax.ShapeDtypeStruct((k, hdim), jnp.bfloat16),
        ],
    )(feat_w, W1)

    bq, bk = 1024, 8192
    nq, nk = n // bq, k // bk
    out2d = pl.pallas_call(
        functools.partial(_main_body, nk=nk),
        grid=(nq, nk),
        in_specs=[
            pl.BlockSpec((bq, c), lambda i, j: (i, 0)),
            pl.BlockSpec((bk, c), lambda i, j: (j, 0)),
            pl.BlockSpec((bk, hdim), lambda i, j: (j, 0)),
            pl.BlockSpec((1, hdim), lambda i, j: (0, 0)),
            pl.BlockSpec((hdim, hdim), lambda i, j: (0, 0)),
            pl.BlockSpec((1, hdim), lambda i, j: (0, 0)),
        ],
        out_specs=pl.BlockSpec((bq, hdim), lambda i, j: (i, 0)),
        out_shape=jax.ShapeDtypeStruct((n, hdim), jnp.float32),
        scratch_shapes=[
            pltpu.VMEM((bq, c), jnp.bfloat16),
            pltpu.VMEM((bq, hdim), jnp.float32),
            pltpu.VMEM((bq, 128), jnp.float32),
        ],
        compiler_params=pltpu.CompilerParams(
            dimension_semantics=("parallel", "arbitrary")),
    )(x_flat, mn, vp, b1_2d, W2, b2_2d)

    return jnp.transpose(out2d.reshape(b, h, w, hdim), (0, 3, 1, 2))


# speedup vs baseline: 1.1441x; 1.0044x over previous
import functools

import jax
import jax.numpy as jnp
from jax.experimental import pallas as pl
from jax.experimental.pallas import tpu as pltpu

_EPS = 1e-12


def _prep_body(fw_ref, w1_ref, mn_ref, vp_ref, *, c):
    fw = fw_ref[...]
    m = fw[:, :c]
    n1 = jnp.sqrt(jnp.sum(m * m, axis=1, keepdims=True))
    mn_ref[...] = (m / jnp.maximum(n1, _EPS)).astype(jnp.bfloat16)
    n2 = jnp.sqrt(jnp.sum(fw * fw, axis=1, keepdims=True))
    fwn = fw / jnp.maximum(n2, _EPS)
    vp_ref[...] = jnp.dot(
        fwn, w1_ref[...], preferred_element_type=jnp.float32
    ).astype(jnp.bfloat16)


def _main_body(xq_ref, mn_ref, vp_ref, b1_ref, w2_ref, b2_ref, out_ref):
    xq = xq_ref[...]
    nrm = jnp.sqrt(jnp.sum(xq * xq, axis=1, keepdims=True))
    xn = (xq / jnp.maximum(nrm, _EPS)).astype(jnp.bfloat16)
    s = jax.lax.dot_general(
        xn, mn_ref[...], (((1,), (1,)), ((), ())),
        preferred_element_type=jnp.float32)
    e = jnp.exp(s)
    acc = jnp.dot(e.astype(jnp.bfloat16), vp_ref[...],
                  preferred_element_type=jnp.float32)
    den = jnp.sum(e, axis=1, keepdims=True)
    o = acc / den + b1_ref[...]
    h1 = 0.5 * o * (1.0 + jax.lax.erf(o * (2.0 ** -0.5)))
    out_ref[...] = (jnp.dot(h1, w2_ref[...], preferred_element_type=jnp.float32)
                    + b2_ref[...])


def kernel(x, feat_w, W1, b1, W2, b2):
    b, c, h, w = x.shape
    n = b * h * w
    k, c4 = feat_w.shape
    hdim = W2.shape[1]

    x_flat = jnp.transpose(x, (0, 2, 3, 1)).reshape(n, c)
    b1_2d = b1.reshape(1, hdim)
    b2_2d = b2.reshape(1, hdim)

    bkp = 1024
    nkp = k // bkp
    mn, vp = pl.pallas_call(
        functools.partial(_prep_body, c=c),
        grid=(nkp,),
        in_specs=[
            pl.BlockSpec((bkp, c4), lambda i: (i, 0)),
            pl.BlockSpec((c4, hdim), lambda i: (0, 0)),
        ],
        out_specs=[
            pl.BlockSpec((bkp, c), lambda i: (i, 0)),
            pl.BlockSpec((bkp, hdim), lambda i: (i, 0)),
        ],
        out_shape=[
            jax.ShapeDtypeStruct((k, c), jnp.bfloat16),
            jax.ShapeDtypeStruct((k, hdim), jnp.bfloat16),
        ],
    )(feat_w, W1)

    bq = 1024
    nq = n // bq
    out2d = pl.pallas_call(
        _main_body,
        grid=(nq,),
        in_specs=[
            pl.BlockSpec((bq, c), lambda i: (i, 0)),
            pl.BlockSpec((k, c), lambda i: (0, 0)),
            pl.BlockSpec((k, hdim), lambda i: (0, 0)),
            pl.BlockSpec((1, hdim), lambda i: (0, 0)),
            pl.BlockSpec((hdim, hdim), lambda i: (0, 0)),
            pl.BlockSpec((1, hdim), lambda i: (0, 0)),
        ],
        out_specs=pl.BlockSpec((bq, hdim), lambda i: (i, 0)),
        out_shape=jax.ShapeDtypeStruct((n, hdim), jnp.float32),
        compiler_params=pltpu.CompilerParams(
            dimension_semantics=("arbitrary",)),
    )(x_flat, mn, vp, b1_2d, W2, b2_2d)

    return jnp.transpose(out2d.reshape(b, h, w, hdim), (0, 3, 1, 2))
